# Initial kernel scaffold; baseline (speedup 1.0000x reference)
import jax
import jax.numpy as jnp
from jax.experimental import pallas as pl


def kernel(x, token_emb, pos_emb):
    raise NotImplementedError("write your pallas kernel here")


# speedup vs baseline: 5.9721x; 5.9721x over previous
import functools

import jax
import jax.numpy as jnp
from jax import lax
from jax.experimental import pallas as pl
from jax.experimental.pallas import tpu as pltpu
from jax.experimental.pallas import tpu_sc as plsc

BATCH = 4096
SEQ = 200
EMBED = 64
N_ROWS = BATCH * SEQ
NUM_WORKERS = 32
ROWS_PER_WORKER = N_ROWS // NUM_WORKERS
CHUNK = 128
NUM_CHUNKS = ROWS_PER_WORKER // CHUNK

_MESH = plsc.VectorSubcoreMesh(
    core_axis_name="c", subcore_axis_name="s", num_cores=2, num_subcores=16
)


@functools.partial(
    pl.kernel,
    out_type=jax.ShapeDtypeStruct((N_ROWS, EMBED), jnp.float32),
    mesh=_MESH,
    compiler_params=pltpu.CompilerParams(use_tc_tiling_on_sc=False),
    scratch_types=[
        pltpu.VMEM((NUM_CHUNKS, CHUNK), jnp.int32),
        pltpu.VMEM((NUM_CHUNKS, CHUNK), jnp.int32),
        pltpu.VMEM((CHUNK, EMBED), jnp.float32),
        pltpu.VMEM((CHUNK, EMBED), jnp.float32),
        pltpu.SemaphoreType.DMA,
        pltpu.SemaphoreType.DMA,
    ],
)
def _embed_sum(notes_hbm, times_hbm, tok_hbm, pos_hbm, out_hbm,
               idx_n, idx_t, buf_n, buf_t, sem_n, sem_t):
    w = lax.axis_index("s") * 2 + lax.axis_index("c")
    pltpu.sync_copy(notes_hbm.at[w], idx_n)
    pltpu.sync_copy(times_hbm.at[w], idx_t)

    def chunk_body(g, carry):
        cp_n = pltpu.async_copy(tok_hbm.at[idx_n.at[g]], buf_n, sem_n)
        cp_t = pltpu.async_copy(pos_hbm.at[idx_t.at[g]], buf_t, sem_t)
        cp_n.wait()
        cp_t.wait()

        def row_body(r, rcarry):
            for cc in range(EMBED // 16):
                sl = pl.ds(cc * 16, 16)
                buf_n[r, sl] = buf_n[r, sl] + buf_t[r, sl]
            return rcarry

        lax.fori_loop(0, CHUNK, row_body, 0)
        pltpu.sync_copy(buf_n, out_hbm.at[pl.ds(w * ROWS_PER_WORKER + g * CHUNK, CHUNK)])
        return carry

    lax.fori_loop(0, NUM_CHUNKS, chunk_body, 0)


def kernel(x, token_emb, pos_emb):
    notes = x[:, 0, :].astype(jnp.int32).reshape(NUM_WORKERS, NUM_CHUNKS, CHUNK)
    times = x[:, 1, :].astype(jnp.int32).reshape(NUM_WORKERS, NUM_CHUNKS, CHUNK)
    out = _embed_sum(notes, times, token_emb, pos_emb)
    return out.reshape(BATCH, SEQ, EMBED)
